# baseline (speedup 1.0000x reference)
import functools

import jax
import jax.numpy as jnp
from jax import lax
from jax.experimental import pallas as pl
from jax.experimental.pallas import tpu as pltpu


def _dot(a, b):
    return lax.dot_general(a, b, (((1,), (0,)), ((), ())),
                           precision=lax.Precision.DEFAULT,
                           preferred_element_type=jnp.float32)


def _step_body(a0_ref, a1_ref, x0_ref, x1_ref, w1_ref, b1_ref, w2_ref,
               b2_ref, o_ref, acc_ref):
    h = _dot(acc_ref[...], w1_ref[...]) + b1_ref[...]
    h = jnp.where(h > 0, h, 0.01 * h)
    o = _dot(h, w2_ref[...]) + b2_ref[...]
    o = jnp.where(o > 0, o, 0.01 * o)
    o_ref[...] = o.astype(o_ref.dtype)
    acc_ref[...] = _dot(a0_ref[...], x0_ref[...]) + _dot(a1_ref[...], x1_ref[...])


def _step(a_f32, x_bf, w1, b1, w2, b2, out_dtype, bm):
    n, k = a_f32.shape
    d = x_bf.shape[1]
    kh = k // 2
    nb = n // bm
    const = lambda i: (0, 0)
    return pl.pallas_call(
        _step_body,
        grid=(nb + 1,),
        in_specs=[
            pl.BlockSpec((bm, kh), lambda i: (jnp.minimum(i, nb - 1), 0)),
            pl.BlockSpec((bm, kh), lambda i: (jnp.minimum(i, nb - 1), 1)),
            pl.BlockSpec((kh, d), const),
            pl.BlockSpec((kh, d), lambda i: (1, 0)),
            pl.BlockSpec(w1.shape, const),
            pl.BlockSpec(b1.shape, const),
            pl.BlockSpec(w2.shape, const),
            pl.BlockSpec(b2.shape, const),
        ],
        out_specs=pl.BlockSpec((bm, d), lambda i: (jnp.maximum(i - 1, 0), 0)),
        out_shape=jax.ShapeDtypeStruct((n, d), out_dtype),
        scratch_shapes=[pltpu.VMEM((bm, d), jnp.float32)],
        compiler_params=pltpu.CompilerParams(
            dimension_semantics=("arbitrary",),
            vmem_limit_bytes=100 * 1024 * 1024,
        ),
    )(a_f32, a_f32, x_bf, x_bf, w1, b1, w2, b2)


def kernel(u_node_feats, v_node_feats, uv_adj_mat, vu_adj_mat,
           W1_uv, b1_uv, W2_uv, b2_uv, W1_vu, b1_vu, W2_vu, b2_vu):
    bf = jnp.bfloat16
    bm = 256
    d = u_node_feats.shape[1]
    u0 = u_node_feats.astype(bf)
    w1u = W1_uv.astype(bf)
    w2u = W2_uv.astype(bf)
    w1v = W1_vu.astype(bf)
    w2v = W2_vu.astype(bf)
    b1u = b1_uv.reshape(1, d)
    b2u = b2_uv.reshape(1, d)
    b1v = b1_vu.reshape(1, d)
    b2v = b2_vu.reshape(1, d)

    v1 = _step(uv_adj_mat, u0, w1u, b1u, w2u, b2u, bf, bm)
    u1 = _step(vu_adj_mat, v1, w1v, b1v, w2v, b2v, bf, bm)
    v2 = _step(uv_adj_mat, u1, w1u, b1u, w2u, b2u, jnp.float32, bm)
    return v2


# speedup vs baseline: 1.0823x; 1.0823x over previous
import functools

import jax
import jax.numpy as jnp
from jax import lax
from jax.experimental import pallas as pl
from jax.experimental.pallas import tpu as pltpu


def _dot(a, b):
    return lax.dot_general(a, b, (((1,), (0,)), ((), ())),
                           precision=lax.Precision.DEFAULT,
                           preferred_element_type=jnp.float32)


def _step_body(a0_ref, a1_ref, x0_ref, x1_ref, w1_ref, b1_ref, w2_ref,
               b2_ref, o_ref, acc_ref):
    h = _dot(acc_ref[...], w1_ref[...]) + b1_ref[...]
    h = jnp.where(h > 0, h, 0.01 * h)
    o = _dot(h, w2_ref[...]) + b2_ref[...]
    o = jnp.where(o > 0, o, 0.01 * o)
    o_ref[...] = o.astype(o_ref.dtype)
    acc_ref[...] = _dot(a0_ref[...], x0_ref[...]) + _dot(a1_ref[...], x1_ref[...])


def _step(a_f32, x_bf, w1, b1, w2, b2, out_dtype, bm):
    n, k = a_f32.shape
    d = x_bf.shape[1]
    kh = k // 2
    nb = n // bm
    const = lambda i: (0, 0)
    return pl.pallas_call(
        _step_body,
        grid=(nb + 1,),
        in_specs=[
            pl.BlockSpec((bm, kh), lambda i: (jnp.minimum(i, nb - 1), 0)),
            pl.BlockSpec((bm, kh), lambda i: (jnp.minimum(i, nb - 1), 1)),
            pl.BlockSpec((kh, d), const),
            pl.BlockSpec((kh, d), lambda i: (1, 0)),
            pl.BlockSpec(w1.shape, const),
            pl.BlockSpec(b1.shape, const),
            pl.BlockSpec(w2.shape, const),
            pl.BlockSpec(b2.shape, const),
        ],
        out_specs=pl.BlockSpec((bm, d), lambda i: (jnp.maximum(i - 1, 0), 0)),
        out_shape=jax.ShapeDtypeStruct((n, d), out_dtype),
        scratch_shapes=[pltpu.VMEM((bm, d), jnp.float32)],
        compiler_params=pltpu.CompilerParams(
            dimension_semantics=("arbitrary",),
            vmem_limit_bytes=100 * 1024 * 1024,
        ),
    )(a_f32, a_f32, x_bf, x_bf, w1, b1, w2, b2)


def kernel(u_node_feats, v_node_feats, uv_adj_mat, vu_adj_mat,
           W1_uv, b1_uv, W2_uv, b2_uv, W1_vu, b1_vu, W2_vu, b2_vu):
    bf = jnp.bfloat16
    bm = 512
    d = u_node_feats.shape[1]
    u0 = u_node_feats.astype(bf)
    w1u = W1_uv.astype(bf)
    w2u = W2_uv.astype(bf)
    w1v = W1_vu.astype(bf)
    w2v = W2_vu.astype(bf)
    b1u = b1_uv.reshape(1, d)
    b2u = b2_uv.reshape(1, d)
    b1v = b1_vu.reshape(1, d)
    b2v = b2_vu.reshape(1, d)

    v1 = _step(uv_adj_mat, u0, w1u, b1u, w2u, b2u, bf, bm)
    u1 = _step(vu_adj_mat, v1, w1v, b1v, w2v, b2v, bf, bm)
    v2 = _step(uv_adj_mat, u1, w1u, b1u, w2u, b2u, jnp.float32, bm)
    return v2
